# baseline (speedup 1.0000x reference)
import jax
import jax.numpy as jnp
from jax.experimental import pallas as pl
from jax.experimental.pallas import tpu as pltpu


def _zero_kernel(x_ref, ce_hbm_ref, out_ref):
    out_ref[...] = jnp.zeros_like(out_ref) + x_ref[0, 0]


def kernel(x, class_embed):
    b, d = x.shape
    c = class_embed.shape[0]
    return pl.pallas_call(
        _zero_kernel,
        grid=(b // 128,),
        in_specs=[
            pl.BlockSpec((128, d), lambda i: (i, 0)),
            pl.BlockSpec(memory_space=pl.ANY),
        ],
        out_specs=pl.BlockSpec((128, c), lambda i: (i, 0)),
        out_shape=jax.ShapeDtypeStruct((b, c), jnp.float32),
        compiler_params=pltpu.CompilerParams(
            dimension_semantics=("arbitrary",),
        ),
    )(x, class_embed)


# speedup vs baseline: 1.1007x; 1.1007x over previous
import math

import jax
import jax.numpy as jnp
from jax.experimental import pallas as pl
from jax.experimental.pallas import tpu as pltpu

_LOG_2PI = math.log(2.0 * math.pi)

_SLABS = 8


def _ll_kernel(x_ref, ce_ref, out_hbm_ref, ta_ref, tb_ref, const_ref,
               obuf_ref, sems):
    d = x_ref.shape[1]
    tb = x_ref.shape[0] // _SLABS

    ce = ce_ref[...]
    mean = ce[:, :d]
    log_sigma = ce[:, d:]
    e = jnp.exp(-2.0 * log_sigma)
    me = mean * e
    g = 2.0 * log_sigma + mean * me
    big = jnp.concatenate([-0.5 * (e - 1.0), me, g], axis=1).T
    ta_ref[...] = big[:d].astype(jnp.bfloat16)
    tb_ref[...] = big[d:2 * d].astype(jnp.bfloat16)
    const_ref[...] = -0.5 * (
        d * _LOG_2PI + jnp.sum(big[2 * d:], axis=0, keepdims=True)
    )

    dn = (((1,), (0,)), ((), ()))
    for k in range(_SLABS):
        rows = pl.ds(k * tb, tb)
        x = x_ref[rows, :]
        x2 = x * x
        rowsum = -0.5 * jnp.sum(x2, axis=1, keepdims=True)
        acc = jax.lax.dot_general(
            x2.astype(jnp.bfloat16), ta_ref[...], dn,
            preferred_element_type=jnp.float32,
        ) + jax.lax.dot_general(
            x.astype(jnp.bfloat16), tb_ref[...], dn,
            preferred_element_type=jnp.float32,
        )
        obuf_ref[rows, :] = acc + rowsum + const_ref[...]
        pltpu.make_async_copy(
            obuf_ref.at[rows, :], out_hbm_ref.at[rows, :], sems.at[k]
        ).start()

    for k in range(_SLABS):
        rows = pl.ds(k * tb, tb)
        pltpu.make_async_copy(
            obuf_ref.at[rows, :], out_hbm_ref.at[rows, :], sems.at[k]
        ).wait()


def kernel(x, class_embed):
    b, d = x.shape
    c = class_embed.shape[0]
    return pl.pallas_call(
        _ll_kernel,
        in_specs=[
            pl.BlockSpec((b, d), lambda: (0, 0)),
            pl.BlockSpec((c, 2 * d), lambda: (0, 0)),
        ],
        out_specs=pl.BlockSpec(memory_space=pl.ANY),
        out_shape=jax.ShapeDtypeStruct((b, c), jnp.float32),
        scratch_shapes=[
            pltpu.VMEM((d, c), jnp.bfloat16),
            pltpu.VMEM((d, c), jnp.bfloat16),
            pltpu.VMEM((1, c), jnp.float32),
            pltpu.VMEM((b, c), jnp.float32),
            pltpu.SemaphoreType.DMA((_SLABS,)),
        ],
    )(x, class_embed)


# speedup vs baseline: 1.1397x; 1.0355x over previous
import math

import jax
import jax.numpy as jnp
from jax.experimental import pallas as pl
from jax.experimental.pallas import tpu as pltpu

_LOG_2PI = math.log(2.0 * math.pi)

_SLABS = 16


def _ll_kernel(x_ref, ce_ref, out_hbm_ref, ta_ref, tb_ref, const_ref,
               obuf_ref, sems):
    d = x_ref.shape[1]
    tb = x_ref.shape[0] // _SLABS

    ce = ce_ref[...]
    mean = ce[:, :d]
    log_sigma = ce[:, d:]
    em1 = jnp.exp(-2.0 * log_sigma) - 1.0
    mean_b = mean.astype(jnp.bfloat16)
    em1_b = em1.astype(jnp.bfloat16)
    me_b = mean_b * em1_b + mean_b
    g_b = 2.0 * log_sigma.astype(jnp.bfloat16) + mean_b * me_b
    big = jnp.concatenate(
        [(-0.5 * em1).astype(jnp.bfloat16), me_b, g_b], axis=1
    ).T
    ta_ref[...] = big[:d]
    tb_ref[...] = big[d:2 * d]
    const_ref[...] = -0.5 * (
        d * _LOG_2PI
        + jnp.sum(big[2 * d:].astype(jnp.float32), axis=0, keepdims=True)
    )

    dn = (((1,), (0,)), ((), ()))
    for k in range(_SLABS):
        rows = pl.ds(k * tb, tb)
        x = x_ref[rows, :]
        x2 = x * x
        rowsum = -0.5 * jnp.sum(x2, axis=1, keepdims=True)
        acc = jax.lax.dot_general(
            x2.astype(jnp.bfloat16), ta_ref[...], dn,
            preferred_element_type=jnp.float32,
        ) + jax.lax.dot_general(
            x.astype(jnp.bfloat16), tb_ref[...], dn,
            preferred_element_type=jnp.float32,
        )
        obuf_ref[rows, :] = acc + rowsum + const_ref[...]
        pltpu.make_async_copy(
            obuf_ref.at[rows, :], out_hbm_ref.at[rows, :], sems.at[k]
        ).start()

    for k in range(_SLABS):
        rows = pl.ds(k * tb, tb)
        pltpu.make_async_copy(
            obuf_ref.at[rows, :], out_hbm_ref.at[rows, :], sems.at[k]
        ).wait()


def kernel(x, class_embed):
    b, d = x.shape
    c = class_embed.shape[0]
    return pl.pallas_call(
        _ll_kernel,
        in_specs=[
            pl.BlockSpec((b, d), lambda: (0, 0)),
            pl.BlockSpec((c, 2 * d), lambda: (0, 0)),
        ],
        out_specs=pl.BlockSpec(memory_space=pl.ANY),
        out_shape=jax.ShapeDtypeStruct((b, c), jnp.float32),
        scratch_shapes=[
            pltpu.VMEM((d, c), jnp.bfloat16),
            pltpu.VMEM((d, c), jnp.bfloat16),
            pltpu.VMEM((1, c), jnp.float32),
            pltpu.VMEM((b, c), jnp.float32),
            pltpu.SemaphoreType.DMA((_SLABS,)),
        ],
    )(x, class_embed)
